# baseline (device time: 110286 ns/iter reference)
import jax
import jax.numpy as jnp
from jax import lax
from jax.experimental import pallas as pl
from jax.experimental.pallas import tpu as pltpu

ROWS = 4096
COLS = 2048
HALF = ROWS // 2
NCHUNK = 16
LAG = 1
CHUNK = HALF // NCHUNK


def kernel(x, pi):
    def body(x_ref, pi_ref, out_ref, f32_buf, bf_half,
             ysend_sems, yrecv_sems, fsend_sems, frecv_sems, copy_sems):
        my_x = lax.axis_index("x")
        my_y = lax.axis_index("y")
        dest_y = pi_ref[my_y]

        def make_cin(row_start, slot):
            return pltpu.make_async_copy(
                x_ref.at[0, pl.ds(row_start, CHUNK), :],
                f32_buf.at[slot],
                copy_sems.at[slot],
            )

        @pl.when(dest_y == my_y)
        def _():
            for c in range(ROWS // CHUNK):
                slot = c % 2
                cin = make_cin(c * CHUNK, slot)
                cin.start()
                cin.wait()
                bf_half[pl.ds(0, CHUNK), :] = (
                    f32_buf[slot, :, :].astype(jnp.bfloat16))
                cout = pltpu.make_async_copy(
                    bf_half.at[pl.ds(0, CHUNK), :],
                    out_ref.at[0, pl.ds(c * CHUNK, CHUNK), :],
                    copy_sems.at[slot],
                )
                cout.start()
                cout.wait()

        @pl.when(dest_y != my_y)
        def _():
            y_peer = (my_x, dest_y)
            x_peer = (1 - my_x, my_y)
            row0 = my_x * HALF
            other0 = (1 - my_x) * HALF

            barrier = pltpu.get_barrier_semaphore()
            for nbr in (y_peer, x_peer):
                pl.semaphore_signal(
                    barrier, inc=1, device_id=nbr,
                    device_id_type=pl.DeviceIdType.MESH,
                )
            pl.semaphore_wait(barrier, 2)

            def forward(c):
                y_rdmas[c].wait_recv()
                rows = pl.ds(row0 + c * CHUNK, CHUNK)
                fwd = pltpu.make_async_remote_copy(
                    src_ref=out_ref.at[0, rows, :],
                    dst_ref=out_ref.at[0, rows, :],
                    send_sem=fsend_sems.at[c],
                    recv_sem=frecv_sems.at[c],
                    device_id=x_peer,
                    device_id_type=pl.DeviceIdType.MESH,
                )
                fwd.start()
                f_rdmas.append(fwd)

            cins = [make_cin(row0 + c * CHUNK, c % 2) for c in range(NCHUNK)]
            cins[0].start()
            y_rdmas = []
            f_rdmas = []
            for c in range(NCHUNK):
                if c + 1 < NCHUNK:
                    cins[c + 1].start()
                cins[c].wait()
                crows = pl.ds(c * CHUNK, CHUNK)
                bf_half[crows, :] = f32_buf[c % 2, :, :].astype(jnp.bfloat16)
                rdma = pltpu.make_async_remote_copy(
                    src_ref=bf_half.at[crows, :],
                    dst_ref=out_ref.at[0, pl.ds(row0 + c * CHUNK, CHUNK), :],
                    send_sem=ysend_sems.at[c],
                    recv_sem=yrecv_sems.at[c],
                    device_id=y_peer,
                    device_id_type=pl.DeviceIdType.MESH,
                )
                rdma.start()
                y_rdmas.append(rdma)
                if c >= LAG:
                    forward(c - LAG)
            for c in range(NCHUNK - LAG, NCHUNK):
                forward(c)

            for c in range(NCHUNK):
                rows = pl.ds(other0 + c * CHUNK, CHUNK)
                recv = pltpu.make_async_remote_copy(
                    src_ref=out_ref.at[0, rows, :],
                    dst_ref=out_ref.at[0, rows, :],
                    send_sem=fsend_sems.at[c],
                    recv_sem=frecv_sems.at[c],
                    device_id=x_peer,
                    device_id_type=pl.DeviceIdType.MESH,
                )
                recv.wait_recv()
            for c in range(NCHUNK):
                y_rdmas[c].wait_send()
                f_rdmas[c].wait_send()

    return pl.pallas_call(
        body,
        out_shape=jax.ShapeDtypeStruct(x.shape, jnp.bfloat16),
        in_specs=[
            pl.BlockSpec(memory_space=pl.ANY),
            pl.BlockSpec(memory_space=pltpu.SMEM),
        ],
        out_specs=pl.BlockSpec(memory_space=pl.ANY),
        scratch_shapes=[
            pltpu.VMEM((2, CHUNK, COLS), jnp.float32),
            pltpu.VMEM((HALF, COLS), jnp.bfloat16),
            pltpu.SemaphoreType.DMA((NCHUNK,)),
            pltpu.SemaphoreType.DMA((NCHUNK,)),
            pltpu.SemaphoreType.DMA((NCHUNK,)),
            pltpu.SemaphoreType.DMA((NCHUNK,)),
            pltpu.SemaphoreType.DMA((2,)),
        ],
        compiler_params=pltpu.CompilerParams(collective_id=0),
    )(x, pi)


# device time: 109935 ns/iter; 1.0032x vs baseline; 1.0032x over previous
import jax
import jax.numpy as jnp
from jax import lax
from jax.experimental import pallas as pl
from jax.experimental.pallas import tpu as pltpu

ROWS = 4096
COLS = 2048
HALF = ROWS // 2
NCHUNK = 16
LAG = 2
CHUNK = HALF // NCHUNK


def kernel(x, pi):
    def body(x_ref, pi_ref, out_ref, f32_buf, bf_half,
             ysend_sems, yrecv_sems, fsend_sems, frecv_sems, copy_sems):
        my_x = lax.axis_index("x")
        my_y = lax.axis_index("y")
        dest_y = pi_ref[my_y]

        def make_cin(row_start, slot):
            return pltpu.make_async_copy(
                x_ref.at[0, pl.ds(row_start, CHUNK), :],
                f32_buf.at[slot],
                copy_sems.at[slot],
            )

        @pl.when(dest_y == my_y)
        def _():
            for c in range(ROWS // CHUNK):
                slot = c % 2
                cin = make_cin(c * CHUNK, slot)
                cin.start()
                cin.wait()
                bf_half[pl.ds(0, CHUNK), :] = (
                    f32_buf[slot, :, :].astype(jnp.bfloat16))
                cout = pltpu.make_async_copy(
                    bf_half.at[pl.ds(0, CHUNK), :],
                    out_ref.at[0, pl.ds(c * CHUNK, CHUNK), :],
                    copy_sems.at[slot],
                )
                cout.start()
                cout.wait()

        @pl.when(dest_y != my_y)
        def _():
            y_peer = (my_x, dest_y)
            x_peer = (1 - my_x, my_y)
            row0 = my_x * HALF
            other0 = (1 - my_x) * HALF

            cins = [make_cin(row0 + c * CHUNK, c % 2) for c in range(NCHUNK)]
            cins[0].start()

            barrier = pltpu.get_barrier_semaphore()
            for nbr in (y_peer, x_peer):
                pl.semaphore_signal(
                    barrier, inc=1, device_id=nbr,
                    device_id_type=pl.DeviceIdType.MESH,
                )
            pl.semaphore_wait(barrier, 2)

            def forward(c):
                y_rdmas[c].wait_recv()
                rows = pl.ds(row0 + c * CHUNK, CHUNK)
                fwd = pltpu.make_async_remote_copy(
                    src_ref=out_ref.at[0, rows, :],
                    dst_ref=out_ref.at[0, rows, :],
                    send_sem=fsend_sems.at[c],
                    recv_sem=frecv_sems.at[c],
                    device_id=x_peer,
                    device_id_type=pl.DeviceIdType.MESH,
                )
                fwd.start()
                f_rdmas.append(fwd)

            y_rdmas = []
            f_rdmas = []
            for c in range(NCHUNK):
                if c + 1 < NCHUNK:
                    cins[c + 1].start()
                cins[c].wait()
                crows = pl.ds(c * CHUNK, CHUNK)
                bf_half[crows, :] = f32_buf[c % 2, :, :].astype(jnp.bfloat16)
                rdma = pltpu.make_async_remote_copy(
                    src_ref=bf_half.at[crows, :],
                    dst_ref=out_ref.at[0, pl.ds(row0 + c * CHUNK, CHUNK), :],
                    send_sem=ysend_sems.at[c],
                    recv_sem=yrecv_sems.at[c],
                    device_id=y_peer,
                    device_id_type=pl.DeviceIdType.MESH,
                )
                rdma.start()
                y_rdmas.append(rdma)
                if c >= LAG:
                    forward(c - LAG)
            for c in range(NCHUNK - LAG, NCHUNK):
                forward(c)

            for c in range(NCHUNK):
                rows = pl.ds(other0 + c * CHUNK, CHUNK)
                recv = pltpu.make_async_remote_copy(
                    src_ref=out_ref.at[0, rows, :],
                    dst_ref=out_ref.at[0, rows, :],
                    send_sem=fsend_sems.at[c],
                    recv_sem=frecv_sems.at[c],
                    device_id=x_peer,
                    device_id_type=pl.DeviceIdType.MESH,
                )
                recv.wait_recv()
            for c in range(NCHUNK):
                y_rdmas[c].wait_send()
                f_rdmas[c].wait_send()

    return pl.pallas_call(
        body,
        out_shape=jax.ShapeDtypeStruct(x.shape, jnp.bfloat16),
        in_specs=[
            pl.BlockSpec(memory_space=pl.ANY),
            pl.BlockSpec(memory_space=pltpu.SMEM),
        ],
        out_specs=pl.BlockSpec(memory_space=pl.ANY),
        scratch_shapes=[
            pltpu.VMEM((2, CHUNK, COLS), jnp.float32),
            pltpu.VMEM((HALF, COLS), jnp.bfloat16),
            pltpu.SemaphoreType.DMA((NCHUNK,)),
            pltpu.SemaphoreType.DMA((NCHUNK,)),
            pltpu.SemaphoreType.DMA((NCHUNK,)),
            pltpu.SemaphoreType.DMA((NCHUNK,)),
            pltpu.SemaphoreType.DMA((2,)),
        ],
        compiler_params=pltpu.CompilerParams(collective_id=0),
    )(x, pi)
